# Initial kernel scaffold; baseline (speedup 1.0000x reference)
import jax
import jax.numpy as jnp
from jax.experimental import pallas as pl


def kernel(x, w_gate, expert_w, expert_b):
    raise NotImplementedError("write your pallas kernel here")


# speedup vs baseline: 2.2862x; 2.2862x over previous
import functools

import jax
import jax.numpy as jnp
from jax import lax
from jax.experimental import pallas as pl
from jax.experimental.pallas import tpu as pltpu
from jax.experimental.pallas import tpu_sc as plsc

E = 64
K = 2
D_IN = 768
D_OUT = 768
T = 2048
P = T * K
TC_CHUNK = 256
N_CHUNKS = T // TC_CHUNK
TILE = 128
N_TILES = P // TILE
NW = 96
SC_CORES = 2
SC_SUBCORES = 16
SC_WORKERS = SC_CORES * SC_SUBCORES


def _gating_kernel(x_ref, wg_ref, pos_ref, gs_ref, counts_ref, loss_ref,
                   e0_s, e1_s, r0_s, r1_s, g0_s, g1_s, carry_s, imp_s, load_s):
    c = pl.program_id(0)

    @pl.when(c == 0)
    def _init():
        carry_s[...] = jnp.zeros_like(carry_s)
        imp_s[...] = jnp.zeros_like(imp_s)
        load_s[...] = jnp.zeros_like(load_s)

    @pl.when(c < N_CHUNKS)
    def _chunk():
        xb = x_ref[...]
        logits = jnp.dot(xb, wg_ref[...],
                         preferred_element_type=jnp.float32)
        eiota = lax.broadcasted_iota(jnp.int32, (TC_CHUNK, E), 1)
        l1 = jnp.max(logits, axis=1, keepdims=True)
        i1 = jnp.min(jnp.where(logits == l1, eiota, E), axis=1, keepdims=True)
        is1 = eiota == i1
        l2 = jnp.max(jnp.where(is1, -jnp.inf, logits), axis=1, keepdims=True)
        i2 = jnp.min(jnp.where((logits == l2) & (~is1), eiota, E),
                     axis=1, keepdims=True)
        is2 = eiota == i2
        sexp = jnp.sum(jnp.exp(logits - l1), axis=1, keepdims=True)
        p1 = 1.0 / sexp
        p2 = jnp.exp(l2 - l1) / sexp
        den = p1 + p2 + 1e-6
        g0 = p1 / den
        g1 = p2 / den
        oh0 = is1.astype(jnp.float32)
        oh1 = is2.astype(jnp.float32)
        oh = oh0 + oh1
        tri = (lax.broadcasted_iota(jnp.int32, (TC_CHUNK, TC_CHUNK), 0) >
               lax.broadcasted_iota(jnp.int32, (TC_CHUNK, TC_CHUNK), 1)
               ).astype(jnp.float32)
        cb = carry_s[...] + jnp.dot(tri, oh, preferred_element_type=jnp.float32,
                                    precision=lax.Precision.HIGHEST)
        r0 = jnp.sum(oh0 * cb, axis=1, keepdims=True)
        r1 = jnp.sum(oh1 * cb, axis=1, keepdims=True)
        carry_s[...] += jnp.sum(oh, axis=0, keepdims=True)
        imp_s[...] += jnp.sum(oh0 * g0 + oh1 * g1, axis=0, keepdims=True)
        load_s[...] += jnp.sum(oh0 * (g0 > 0.0).astype(jnp.float32) +
                               oh1 * (g1 > 0.0).astype(jnp.float32),
                               axis=0, keepdims=True)
        sl = pl.ds(c * TC_CHUNK, TC_CHUNK)
        e0_s[sl, :] = i1
        e1_s[sl, :] = i2
        r0_s[sl, :] = r0
        r1_s[sl, :] = r1
        g0_s[sl, :] = g0
        g1_s[sl, :] = g1

    @pl.when(c == N_CHUNKS)
    def _final():
        counts = carry_s[...]
        up = (lax.broadcasted_iota(jnp.int32, (E, E), 0) <
              lax.broadcasted_iota(jnp.int32, (E, E), 1)).astype(jnp.float32)
        off = jnp.dot(counts, up, preferred_element_type=jnp.float32,
                      precision=lax.Precision.HIGHEST)
        eiota = lax.broadcasted_iota(jnp.int32, (T, E), 1)
        off0 = jnp.sum(jnp.where(e0_s[...] == eiota, off, 0.0),
                       axis=1, keepdims=True)
        off1 = jnp.sum(jnp.where(e1_s[...] == eiota, off, 0.0),
                       axis=1, keepdims=True)
        pos0 = (off0 + r0_s[...]).astype(jnp.int32)
        pos1 = (off1 + r1_s[...]).astype(jnp.int32)
        pos_ref[:, 0:1] = pos0
        pos_ref[:, 1:2] = pos1
        counts_ref[...] = counts
        for sc in range(P // 512):
            siota = sc * 512 + lax.broadcasted_iota(jnp.int32, (T, 512), 1)
            gsc = jnp.sum(jnp.where(pos0 == siota, g0_s[...], 0.0) +
                          jnp.where(pos1 == siota, g1_s[...], 0.0),
                          axis=0, keepdims=True)
            gs_ref[:, pl.ds(sc * 512, 512)] = gsc

        def cv2(v):
            m = jnp.sum(v, axis=1, keepdims=True) / E
            var = jnp.sum((v - m) ** 2, axis=1, keepdims=True) / (E - 1)
            return var / (m * m + 1e-10)

        loss_ref[...] = (cv2(imp_s[...]) + cv2(load_s[...])) * 1e-2


def _run_gating(x, w_gate):
    return pl.pallas_call(
        _gating_kernel,
        grid=(N_CHUNKS + 1,),
        in_specs=[
            pl.BlockSpec((TC_CHUNK, D_IN),
                         lambda c: (jnp.minimum(c, N_CHUNKS - 1), 0)),
            pl.BlockSpec((D_IN, E), lambda c: (0, 0)),
        ],
        out_specs=[
            pl.BlockSpec((T, K), lambda c: (0, 0)),
            pl.BlockSpec((1, P), lambda c: (0, 0)),
            pl.BlockSpec((1, E), lambda c: (0, 0)),
            pl.BlockSpec((1, 1), lambda c: (0, 0)),
        ],
        out_shape=[
            jax.ShapeDtypeStruct((T, K), jnp.int32),
            jax.ShapeDtypeStruct((1, P), jnp.float32),
            jax.ShapeDtypeStruct((1, E), jnp.float32),
            jax.ShapeDtypeStruct((1, 1), jnp.float32),
        ],
        scratch_shapes=[
            pltpu.VMEM((T, 1), jnp.int32),
            pltpu.VMEM((T, 1), jnp.int32),
            pltpu.VMEM((T, 1), jnp.float32),
            pltpu.VMEM((T, 1), jnp.float32),
            pltpu.VMEM((T, 1), jnp.float32),
            pltpu.VMEM((T, 1), jnp.float32),
            pltpu.VMEM((1, E), jnp.float32),
            pltpu.VMEM((1, E), jnp.float32),
            pltpu.VMEM((1, E), jnp.float32),
        ],
    )(x, w_gate)


def _dispatch_body(x_hbm, pos_hbm, xs_hbm, tok_v, pos_v, rows_v, sem_g, sem_s):
    wid = lax.axis_index("s") * SC_CORES + lax.axis_index("c")
    base = wid * (P // SC_WORKERS)
    pltpu.sync_copy(pos_hbm.at[pl.ds(base, P // SC_WORKERS)], pos_v)
    ii = lax.iota(jnp.int32, 16)
    for cth in range((P // SC_WORKERS) // 16):
        tok_v[pl.ds(cth * 16, 16)] = (base + cth * 16 + ii) >> 1
    pltpu.async_copy(x_hbm.at[tok_v], rows_v, sem_g).wait()
    pltpu.async_copy(rows_v, xs_hbm.at[pos_v], sem_s).wait()


def _run_dispatch(x, pos_flat):
    mesh = plsc.VectorSubcoreMesh(core_axis_name="c", subcore_axis_name="s",
                                  num_cores=SC_CORES, num_subcores=SC_SUBCORES)
    npw = P // SC_WORKERS
    f = pl.kernel(
        _dispatch_body,
        out_type=jax.ShapeDtypeStruct((P, D_IN), jnp.float32),
        mesh=mesh,
        scratch_types=[
            pltpu.VMEM((npw,), jnp.int32),
            pltpu.VMEM((npw,), jnp.int32),
            pltpu.VMEM((npw, D_IN), jnp.float32),
            pltpu.SemaphoreType.DMA,
            pltpu.SemaphoreType.DMA,
        ],
    )
    return f(x, pos_flat)


def _expert_mm_kernel(tile_ref, expert_ref, lo_ref, hi_ref,
                      xs_ref, w_ref, b_ref, gs_ref, out_ref):
    w = pl.program_id(0)
    tile = tile_ref[w]
    rel_lo = lo_ref[w] - tile * TILE
    rel_hi = hi_ref[w] - tile * TILE
    rio = lax.broadcasted_iota(jnp.int32, (TILE, 1), 0)
    active = (rio >= rel_lo) & (rio < rel_hi)
    xm = jnp.where(active, xs_ref[...], 0.0)
    g = gs_ref[...]
    z = g * jnp.dot(xm, w_ref[0], preferred_element_type=jnp.float32)
    z = z + jnp.where(active, g * b_ref[0], 0.0)
    first = jnp.logical_or(w == 0,
                           tile_ref[jnp.maximum(w - 1, 0)] != tile)

    @pl.when(first)
    def _set():
        out_ref[...] = z

    @pl.when(jnp.logical_not(first))
    def _acc():
        out_ref[...] += z


def _run_expert_mm(xs, expert_w, expert_b, gs,
                   wi_tile, wi_expert, wi_lo, wi_hi):
    grid_spec = pltpu.PrefetchScalarGridSpec(
        num_scalar_prefetch=4,
        grid=(NW,),
        in_specs=[
            pl.BlockSpec((TILE, D_IN), lambda w, t, e, lo, hi: (t[w], 0)),
            pl.BlockSpec((1, D_IN, D_OUT), lambda w, t, e, lo, hi: (e[w], 0, 0)),
            pl.BlockSpec((1, 1, D_OUT), lambda w, t, e, lo, hi: (e[w], 0, 0)),
            pl.BlockSpec((TILE, 1), lambda w, t, e, lo, hi: (t[w], 0)),
        ],
        out_specs=pl.BlockSpec((TILE, D_OUT), lambda w, t, e, lo, hi: (t[w], 0)),
    )
    return pl.pallas_call(
        _expert_mm_kernel,
        grid_spec=grid_spec,
        out_shape=jax.ShapeDtypeStruct((P, D_OUT), jnp.float32),
    )(wi_tile, wi_expert, wi_lo, wi_hi, xs, expert_w,
      expert_b.reshape(E, 1, D_OUT), gs.reshape(P, 1))


def _combine_body(pout_hbm, pos0_hbm, pos1_hbm, y_hbm,
                  idx0_v, idx1_v, r0_v, r1_v, y_v, sem0, sem1):
    wid = lax.axis_index("s") * SC_CORES + lax.axis_index("c")
    tpw = T // SC_WORKERS
    for chunk in range(2):
        base = wid * tpw + chunk * 32
        pltpu.sync_copy(pos0_hbm.at[pl.ds(base, 32)], idx0_v)
        pltpu.sync_copy(pos1_hbm.at[pl.ds(base, 32)], idx1_v)
        cp0 = pltpu.async_copy(pout_hbm.at[idx0_v], r0_v, sem0)
        cp1 = pltpu.async_copy(pout_hbm.at[idx1_v], r1_v, sem1)
        cp0.wait()
        cp1.wait()

        def body(i, carry):
            for c in range(D_OUT // 16):
                sl = pl.ds(c * 16, 16)
                y_v[i, sl] = r0_v[i, sl] + r1_v[i, sl]
            return carry

        lax.fori_loop(0, 32, body, 0)
        pltpu.sync_copy(y_v, y_hbm.at[pl.ds(base, 32)])


def _run_combine(pout, pos0, pos1):
    mesh = plsc.VectorSubcoreMesh(core_axis_name="c", subcore_axis_name="s",
                                  num_cores=SC_CORES, num_subcores=SC_SUBCORES)
    f = pl.kernel(
        _combine_body,
        out_type=jax.ShapeDtypeStruct((T, D_OUT), jnp.float32),
        mesh=mesh,
        scratch_types=[
            pltpu.VMEM((32,), jnp.int32),
            pltpu.VMEM((32,), jnp.int32),
            pltpu.VMEM((32, D_OUT), jnp.float32),
            pltpu.VMEM((32, D_OUT), jnp.float32),
            pltpu.VMEM((32, D_OUT), jnp.float32),
            pltpu.SemaphoreType.DMA,
            pltpu.SemaphoreType.DMA,
        ],
    )
    return f(pout, pos0, pos1)


def _work_items(counts):
    counts_i = counts[0].astype(jnp.int32)
    off = jnp.concatenate([jnp.zeros((1,), jnp.int32),
                           jnp.cumsum(counts_i)])
    tile_starts = jnp.arange(N_TILES, dtype=jnp.int32) * TILE
    e_lo = jnp.sum(off[None, :E] <= tile_starts[:, None], axis=1) - 1
    e_hi = jnp.sum(off[None, :E] <= (tile_starts + TILE - 1)[:, None],
                   axis=1) - 1
    n_i = e_hi - e_lo + 1
    cumt = jnp.concatenate([jnp.zeros((1,), jnp.int32), jnp.cumsum(n_i)])
    total = cumt[N_TILES]
    w = jnp.arange(NW, dtype=jnp.int32)
    i_w = jnp.sum(cumt[None, :N_TILES] <= w[:, None], axis=1) - 1
    e_w = jnp.clip(e_lo[i_w] + (w - cumt[i_w]), 0, E - 1)
    valid = w < total
    e_last = e_hi[N_TILES - 1]
    e_w = jnp.where(valid, e_w, e_last)
    tile_w = jnp.where(valid, i_w, N_TILES - 1)
    lo_w = jnp.where(valid, jnp.maximum(off[e_w], tile_w * TILE), 0)
    hi_w = jnp.where(valid, jnp.minimum(off[e_w + 1], tile_w * TILE + TILE), 0)
    return tile_w, e_w, lo_w, hi_w


def kernel(x, w_gate, expert_w, expert_b):
    pos, gs, counts, loss = _run_gating(x, w_gate)
    pos_flat = pos.reshape(P)
    wi_tile, wi_expert, wi_lo, wi_hi = _work_items(counts)
    xs = _run_dispatch(x, pos_flat)
    pout = _run_expert_mm(xs, expert_w, expert_b, gs,
                          wi_tile, wi_expert, wi_lo, wi_hi)
    y = _run_combine(pout, pos[:, 0].copy(), pos[:, 1].copy())
    return y, loss[0, 0]


# speedup vs baseline: 2.3048x; 1.0082x over previous
import jax
import jax.numpy as jnp
from jax import lax
from jax.experimental import pallas as pl
from jax.experimental.pallas import tpu as pltpu
from jax.experimental.pallas import tpu_sc as plsc

E = 64
K = 2
D_IN = 768
D_OUT = 768
T = 2048
P = T * K
TILE = 128
N_TILES = P // TILE
NW = 96
SC_CORES = 2
SC_SUBCORES = 16
SC_WORKERS = SC_CORES * SC_SUBCORES


def _gating_kernel(x_ref, wg_ref, pos_ref, pos0_ref, pos1_ref, gs_ref,
                   counts_ref, loss_ref):
    logits = jnp.dot(x_ref[...], wg_ref[...],
                     preferred_element_type=jnp.float32)
    eiota = lax.broadcasted_iota(jnp.int32, (T, E), 1)
    l1 = jnp.max(logits, axis=1, keepdims=True)
    i1 = jnp.min(jnp.where(logits == l1, eiota, E), axis=1, keepdims=True)
    is1 = eiota == i1
    l2 = jnp.max(jnp.where(is1, -jnp.inf, logits), axis=1, keepdims=True)
    i2 = jnp.min(jnp.where((logits == l2) & (~is1), eiota, E),
                 axis=1, keepdims=True)
    is2 = eiota == i2
    sexp = jnp.sum(jnp.exp(logits - l1), axis=1, keepdims=True)
    p1 = 1.0 / sexp
    p2 = jnp.exp(l2 - l1) / sexp
    den = p1 + p2 + 1e-6
    g0 = p1 / den
    g1 = p2 / den
    oh0 = is1.astype(jnp.float32)
    oh1 = is2.astype(jnp.float32)
    oh = oh0 + oh1
    tri = (lax.broadcasted_iota(jnp.int32, (T, T), 0) >
           lax.broadcasted_iota(jnp.int32, (T, T), 1)).astype(jnp.bfloat16)
    cb = jnp.dot(tri, oh.astype(jnp.bfloat16),
                 preferred_element_type=jnp.float32)
    r0 = jnp.sum(oh0 * cb, axis=1, keepdims=True)
    r1 = jnp.sum(oh1 * cb, axis=1, keepdims=True)
    counts = jnp.sum(oh, axis=0, keepdims=True)
    imp = jnp.sum(oh0 * g0 + oh1 * g1, axis=0, keepdims=True)
    load = jnp.sum(oh0 * (g0 > 0.0).astype(jnp.float32) +
                   oh1 * (g1 > 0.0).astype(jnp.float32), axis=0, keepdims=True)
    up = (lax.broadcasted_iota(jnp.int32, (E, E), 0) <
          lax.broadcasted_iota(jnp.int32, (E, E), 1)).astype(jnp.float32)
    off = jnp.dot(counts, up, preferred_element_type=jnp.float32,
                  precision=lax.Precision.HIGHEST)
    off0 = jnp.sum(jnp.where(is1, off, 0.0), axis=1, keepdims=True)
    off1 = jnp.sum(jnp.where(is2, off, 0.0), axis=1, keepdims=True)
    pos0 = (off0 + r0).astype(jnp.int32)
    pos1 = (off1 + r1).astype(jnp.int32)
    pos_ref[:, 0:1] = pos0
    pos_ref[:, 1:2] = pos1
    pos0_ref[...] = pos0
    pos1_ref[...] = pos1
    counts_ref[...] = counts
    for sc in range(P // 512):
        siota = sc * 512 + lax.broadcasted_iota(jnp.int32, (T, 512), 1)
        gsc = jnp.sum(jnp.where(pos0 == siota, g0, 0.0) +
                      jnp.where(pos1 == siota, g1, 0.0),
                      axis=0, keepdims=True)
        gs_ref[:, pl.ds(sc * 512, 512)] = gsc

    def cv2(v):
        m = jnp.sum(v, axis=1, keepdims=True) / E
        var = jnp.sum((v - m) ** 2, axis=1, keepdims=True) / (E - 1)
        return var / (m * m + 1e-10)

    loss_ref[...] = (cv2(imp) + cv2(load)) * 1e-2


def _run_gating(x, w_gate):
    return pl.pallas_call(
        _gating_kernel,
        out_shape=[
            jax.ShapeDtypeStruct((T, K), jnp.int32),
            jax.ShapeDtypeStruct((T, 1), jnp.int32),
            jax.ShapeDtypeStruct((T, 1), jnp.int32),
            jax.ShapeDtypeStruct((1, P), jnp.float32),
            jax.ShapeDtypeStruct((1, E), jnp.float32),
            jax.ShapeDtypeStruct((1, 1), jnp.float32),
        ],
    )(x, w_gate)


def _dispatch_body(x_hbm, pos_hbm, xs_hbm, tok_v, pos_v, rows_v, sem_g, sem_s):
    wid = lax.axis_index("s") * SC_CORES + lax.axis_index("c")
    base = wid * (P // SC_WORKERS)
    pltpu.sync_copy(pos_hbm.at[pl.ds(base, P // SC_WORKERS)], pos_v)
    ii = lax.iota(jnp.int32, 16)
    for cth in range((P // SC_WORKERS) // 16):
        tok_v[pl.ds(cth * 16, 16)] = (base + cth * 16 + ii) >> 1
    pltpu.async_copy(x_hbm.at[tok_v], rows_v, sem_g).wait()
    pltpu.async_copy(rows_v, xs_hbm.at[pos_v], sem_s).wait()


def _run_dispatch(x, pos_flat):
    mesh = plsc.VectorSubcoreMesh(core_axis_name="c", subcore_axis_name="s",
                                  num_cores=SC_CORES, num_subcores=SC_SUBCORES)
    npw = P // SC_WORKERS
    f = pl.kernel(
        _dispatch_body,
        out_type=jax.ShapeDtypeStruct((P, D_IN), jnp.float32),
        mesh=mesh,
        scratch_types=[
            pltpu.VMEM((npw,), jnp.int32),
            pltpu.VMEM((npw,), jnp.int32),
            pltpu.VMEM((npw, D_IN), jnp.float32),
            pltpu.SemaphoreType.DMA,
            pltpu.SemaphoreType.DMA,
        ],
    )
    return f(x, pos_flat)


def _expert_mm_kernel(tile_ref, expert_ref, lo_ref, hi_ref,
                      xs_ref, w_ref, b_ref, gs_ref, out_ref):
    w = pl.program_id(0)
    tile = tile_ref[w]
    lo = lo_ref[w]
    hi = hi_ref[w]

    @pl.when(w == 0)
    def _zero():
        out_ref[...] = jnp.zeros_like(out_ref)

    @pl.when(hi > lo)
    def _compute():
        rel_lo = lo - tile * TILE
        rel_hi = hi - tile * TILE
        rio = lax.broadcasted_iota(jnp.int32, (TILE, 1), 0)
        active = (rio >= rel_lo) & (rio < rel_hi)
        sl = pl.ds(tile * TILE, TILE)
        xm = jnp.where(active, xs_ref[sl, :], 0.0)
        g = gs_ref[sl, :]
        z = g * jnp.dot(xm.astype(jnp.bfloat16),
                        w_ref[0].astype(jnp.bfloat16),
                        preferred_element_type=jnp.float32)
        z = z + jnp.where(active, g * b_ref[0], 0.0)
        out_ref[sl, :] += z


def _run_expert_mm(xs, expert_w, expert_b, gs,
                   wi_tile, wi_expert, wi_lo, wi_hi):
    grid_spec = pltpu.PrefetchScalarGridSpec(
        num_scalar_prefetch=4,
        grid=(NW,),
        in_specs=[
            pl.BlockSpec((P, D_IN), lambda w, t, e, lo, hi: (0, 0)),
            pl.BlockSpec((1, D_IN, D_OUT), lambda w, t, e, lo, hi: (e[w], 0, 0)),
            pl.BlockSpec((1, 1, D_OUT), lambda w, t, e, lo, hi: (e[w], 0, 0)),
            pl.BlockSpec((P, 1), lambda w, t, e, lo, hi: (0, 0)),
        ],
        out_specs=pl.BlockSpec((P, D_OUT), lambda w, t, e, lo, hi: (0, 0)),
    )
    return pl.pallas_call(
        _expert_mm_kernel,
        grid_spec=grid_spec,
        out_shape=jax.ShapeDtypeStruct((P, D_OUT), jnp.float32),
    )(wi_tile, wi_expert, wi_lo, wi_hi, xs, expert_w,
      expert_b.reshape(E, 1, D_OUT), gs.reshape(P, 1))


def _combine_body(pout_hbm, pos0_hbm, pos1_hbm, y_hbm,
                  idx0_v, idx1_v, r0_v, r1_v, y_v, sem0, sem1):
    wid = lax.axis_index("s") * SC_CORES + lax.axis_index("c")
    tpw = T // SC_WORKERS
    for chunk in range(2):
        base = wid * tpw + chunk * 32
        pltpu.sync_copy(pos0_hbm.at[pl.ds(base, 32)], idx0_v)
        pltpu.sync_copy(pos1_hbm.at[pl.ds(base, 32)], idx1_v)
        cp0 = pltpu.async_copy(pout_hbm.at[idx0_v], r0_v, sem0)
        cp1 = pltpu.async_copy(pout_hbm.at[idx1_v], r1_v, sem1)
        cp0.wait()
        cp1.wait()

        def body(i, carry):
            for c in range(D_OUT // 16):
                sl = pl.ds(c * 16, 16)
                y_v[i, sl] = r0_v[i, sl] + r1_v[i, sl]
            return carry

        lax.fori_loop(0, 32, body, 0)
        pltpu.sync_copy(y_v, y_hbm.at[pl.ds(base, 32)])


def _run_combine(pout, pos0, pos1):
    mesh = plsc.VectorSubcoreMesh(core_axis_name="c", subcore_axis_name="s",
                                  num_cores=SC_CORES, num_subcores=SC_SUBCORES)
    f = pl.kernel(
        _combine_body,
        out_type=jax.ShapeDtypeStruct((T, D_OUT), jnp.float32),
        mesh=mesh,
        scratch_types=[
            pltpu.VMEM((32,), jnp.int32),
            pltpu.VMEM((32,), jnp.int32),
            pltpu.VMEM((32, D_OUT), jnp.float32),
            pltpu.VMEM((32, D_OUT), jnp.float32),
            pltpu.VMEM((32, D_OUT), jnp.float32),
            pltpu.SemaphoreType.DMA,
            pltpu.SemaphoreType.DMA,
        ],
    )
    return f(pout, pos0, pos1)


def _work_items(counts):
    counts_i = counts[0].astype(jnp.int32)
    off = jnp.concatenate([jnp.zeros((1,), jnp.int32),
                           jnp.cumsum(counts_i)])
    tile_lo = off[:E] // TILE
    tile_hi = (off[1:] - 1) // TILE
    n_e = jnp.where(counts_i > 0, tile_hi - tile_lo + 1, 0)
    cum = jnp.concatenate([jnp.zeros((1,), jnp.int32), jnp.cumsum(n_e)])
    total = cum[E]
    w = jnp.arange(NW, dtype=jnp.int32)
    e_w = jnp.sum(cum[None, :E] <= w[:, None], axis=1) - 1
    e_w = jnp.clip(e_w, 0, E - 1)
    tile_w = jnp.clip(tile_lo[e_w] + (w - cum[e_w]), 0, N_TILES - 1)
    e_last = jnp.sum(cum[:E] <= total - 1) - 1
    tile_last = tile_lo[e_last] + (total - 1 - cum[e_last])
    valid = w < total
    e_w = jnp.where(valid, e_w, e_last)
    tile_w = jnp.where(valid, tile_w, tile_last)
    lo_w = jnp.where(valid, jnp.maximum(off[e_w], tile_w * TILE), 0)
    hi_w = jnp.where(valid, jnp.minimum(off[e_w + 1], tile_w * TILE + TILE), 0)
    return tile_w, e_w, lo_w, hi_w


def kernel(x, w_gate, expert_w, expert_b):
    pos, pos0, pos1, gs, counts, loss = _run_gating(x, w_gate)
    pos_flat = pos.reshape(P)
    wi_tile, wi_expert, wi_lo, wi_hi = _work_items(counts)
    xs = _run_dispatch(x, pos_flat)
    pout = _run_expert_mm(xs, expert_w, expert_b, gs,
                          wi_tile, wi_expert, wi_lo, wi_hi)
    y = _run_combine(pout, pos0.reshape(T), pos1.reshape(T))
    return y, loss[0, 0]
